# Initial kernel scaffold; baseline (speedup 1.0000x reference)
import jax
import jax.numpy as jnp
from jax.experimental import pallas as pl


def kernel(x, edge_index, edge_attr, u, acting_mask, params):
    raise NotImplementedError("write your pallas kernel here")


# speedup vs baseline: 28.9893x; 28.9893x over previous
import functools

import jax
import jax.numpy as jnp
from jax import lax
from jax.experimental import pallas as pl
from jax.experimental.pallas import tpu as pltpu
from jax.experimental.pallas import tpu_sc as plsc

F32 = jnp.float32

H = 4
C = 32
HID = 128
N_EDGE = 4
NEG = 0.2

N = 10000
E = 320000

NC = 2
NS = 16
NW = NC * NS
EW = E // NW
R = 80
K = EW // R

BN = 2000
BE = 2560


def _ln(x, g, b):
    m = jnp.mean(x, axis=-1, keepdims=True)
    v = jnp.mean((x - m) ** 2, axis=-1, keepdims=True)
    return (x - m) * jax.lax.rsqrt(v + 1e-5) * g + b


def _gelu(x):
    return 0.5 * x * (1.0 + lax.erf(x * 0.7071067811865476))


def _softplus(x):
    return jnp.maximum(x, 0.0) + jnp.log1p(jnp.exp(-jnp.abs(x)))


def _make_gather2():
    mesh = plsc.VectorSubcoreMesh(core_axis_name="c", subcore_axis_name="s")

    @functools.partial(
        pl.kernel,
        mesh=mesh,
        out_type=(
            jax.ShapeDtypeStruct((E, HID), F32),
            jax.ShapeDtypeStruct((E, HID), F32),
        ),
        scratch_types=[
            pltpu.VMEM((K, R), jnp.int32),
            pltpu.VMEM((K, R), jnp.int32),
            pltpu.VMEM((R, HID), F32),
            pltpu.VMEM((R, HID), F32),
            pltpu.SemaphoreType.DMA,
        ],
    )
    def gather2(tab_a, tab_b, idx_a, idx_b, out_a, out_b,
                idxa_v, idxb_v, buf_a, buf_b, sem):
        cid = lax.axis_index("c")
        sid = lax.axis_index("s")
        wid = sid * NC + cid
        pltpu.sync_copy(idx_a.at[wid], idxa_v)
        pltpu.sync_copy(idx_b.at[wid], idxb_v)

        def chunk(j, carry):
            e0 = wid * EW + j * R
            pltpu.async_copy(tab_a.at[idxa_v.at[j]], buf_a, sem).wait()
            pltpu.sync_copy(buf_a, out_a.at[pl.ds(e0, R), :])
            pltpu.async_copy(tab_b.at[idxb_v.at[j]], buf_b, sem).wait()
            pltpu.sync_copy(buf_b, out_b.at[pl.ds(e0, R), :])
            return carry

        lax.fori_loop(0, K, chunk, 0)

    return gather2


def _make_scatter(width):
    mesh = plsc.VectorSubcoreMesh(core_axis_name="c", subcore_axis_name="s")
    rps = 624
    tail0 = rps * NS

    @functools.partial(
        pl.kernel,
        mesh=mesh,
        out_type=jax.ShapeDtypeStruct((NC, N, width), F32),
        scratch_types=[
            pltpu.VMEM((K, R), jnp.int32),
            pltpu.VMEM((R, width), F32),
            pltpu.VMEM_SHARED((N, width), F32),
            pltpu.SemaphoreType.DMA,
        ],
    )
    def scatter(vals, idx, zeros, out, idx_v, buf, acc, sem):
        cid = lax.axis_index("c")
        sid = lax.axis_index("s")
        wid = sid * NC + cid
        r0 = sid * rps
        pltpu.sync_copy(zeros.at[pl.ds(r0, rps), :],
                        acc.at[pl.ds(r0, rps), :])

        @pl.when(sid == NS - 1)
        def _init_tail():
            pltpu.sync_copy(zeros.at[pl.ds(tail0, N - tail0), :],
                            acc.at[pl.ds(tail0, N - tail0), :])

        plsc.subcore_barrier()
        pltpu.sync_copy(idx.at[wid], idx_v)

        def chunk(j, carry):
            e0 = wid * EW + j * R
            pltpu.sync_copy(vals.at[pl.ds(e0, R), :], buf)
            pltpu.sync_copy(buf, acc.at[idx_v.at[j]], add=True)
            return carry

        lax.fori_loop(0, K, chunk, 0)
        plsc.subcore_barrier()
        pltpu.sync_copy(acc.at[pl.ds(r0, rps), :],
                        out.at[cid, pl.ds(r0, rps), :])

        @pl.when(sid == NS - 1)
        def _drain_tail():
            pltpu.sync_copy(acc.at[pl.ds(tail0, N - tail0), :],
                            out.at[cid, pl.ds(tail0, N - tail0), :])

    return scatter


_SC_CACHE = {}


def _gather2(tab_a, tab_b, idx_a, idx_b):
    if 'g2' not in _SC_CACHE:
        _SC_CACHE['g2'] = _make_gather2()
    return _SC_CACHE['g2'](tab_a, tab_b, idx_a, idx_b)


def _scatter128(vals, idx):
    if 's128' not in _SC_CACHE:
        _SC_CACHE['s128'] = _make_scatter(HID)
    return _SC_CACHE['s128'](vals, idx, jnp.zeros((N, HID), F32))


def _full(shape):
    nd = len(shape)
    return pl.BlockSpec(shape, lambda i: (0,) * nd)


def _enc_kernel(x, uu, ne_w, ne_b, ne_g, ne_bt, ge_w, ge_b, ge_g, ge_bt, out):
    t = jnp.dot(x[:], ne_w[:], preferred_element_type=F32) + ne_b[:]
    t = _gelu(_ln(t, ne_g[:], ne_bt[:]))
    tu = jnp.dot(uu[:], ge_w[:], preferred_element_type=F32) + ge_b[:]
    tu = _gelu(_ln(tu, ge_g[:], ge_bt[:]))
    out[:] = t + tu


def _encoder(x, uu, p):
    return pl.pallas_call(
        _enc_kernel,
        grid=(N // BN,),
        in_specs=[
            pl.BlockSpec((BN, 18), lambda i: (i, 0)),
            _full((1, 8)), _full((18, HID)), _full((1, HID)), _full((1, HID)),
            _full((1, HID)), _full((8, HID)), _full((1, HID)), _full((1, HID)),
            _full((1, HID)),
        ],
        out_specs=pl.BlockSpec((BN, HID), lambda i: (i, 0)),
        out_shape=jax.ShapeDtypeStruct((N, HID), F32),
    )(x, uu, p['ne_w'], p['ne_b'].reshape(1, -1), p['ne_g'].reshape(1, -1),
      p['ne_bt'].reshape(1, -1), p['ge_w'], p['ge_b'].reshape(1, -1),
      p['ge_g'].reshape(1, -1), p['ge_bt'].reshape(1, -1))


def _lin2_kernel(h, wl, bl, wr, br, xl, xr):
    hh = h[:]
    xl[:] = jnp.dot(hh, wl[:], preferred_element_type=F32) + bl[:]
    xr[:] = jnp.dot(hh, wr[:], preferred_element_type=F32) + br[:]


def _lin2(h, wl, bl, wr, br):
    return pl.pallas_call(
        _lin2_kernel,
        grid=(N // BN,),
        in_specs=[
            pl.BlockSpec((BN, HID), lambda i: (i, 0)),
            _full((HID, HID)), _full((1, HID)),
            _full((HID, HID)), _full((1, HID)),
        ],
        out_specs=[
            pl.BlockSpec((BN, HID), lambda i: (i, 0)),
            pl.BlockSpec((BN, HID), lambda i: (i, 0)),
        ],
        out_shape=[
            jax.ShapeDtypeStruct((N, HID), F32),
            jax.ShapeDtypeStruct((N, HID), F32),
        ],
    )(h, wl, bl.reshape(1, -1), wr, br.reshape(1, -1))


def _edge_kernel(gl, gr, ea, we, att16, hexp, w_out, a_out):
    glv = gl[:]
    ee = jnp.dot(ea[:], we[:], preferred_element_type=F32)
    xe = glv + gr[:] + ee
    lr = jnp.maximum(xe, NEG * xe)
    a16 = jnp.exp(jnp.dot(lr, att16[:], preferred_element_type=F32))
    w_out[:] = glv * jnp.dot(a16, hexp[:], preferred_element_type=F32)
    a_out[:] = jnp.concatenate([a16, jnp.zeros((BE, HID - 16), F32)], axis=1)


def _edge_stage(gl, gr, ea, we, att16, hexp):
    return pl.pallas_call(
        _edge_kernel,
        grid=(E // BE,),
        in_specs=[
            pl.BlockSpec((BE, HID), lambda i: (i, 0)),
            pl.BlockSpec((BE, HID), lambda i: (i, 0)),
            pl.BlockSpec((BE, N_EDGE), lambda i: (i, 0)),
            _full((N_EDGE, HID)), _full((HID, 16)), _full((16, HID)),
        ],
        out_specs=[
            pl.BlockSpec((BE, HID), lambda i: (i, 0)),
            pl.BlockSpec((BE, HID), lambda i: (i, 0)),
        ],
        out_shape=[
            jax.ShapeDtypeStruct((E, HID), F32),
            jax.ShapeDtypeStruct((E, HID), F32),
        ],
    )(gl, gr, ea, we, att16, hexp)


def _combine_kernel(h, xl, xr, nm0, nm1, dn0, dn1, dg0, dg1, we, att16, hexp,
                    bias, g, b, out):
    hv = h[:]
    xlv = xl[:]
    dl = dg0[:] + dg1[:]
    deg = jnp.maximum(dl[:, 4:5], 1.0)
    loop_attr = dl[:, 0:4] / deg
    lee = jnp.dot(loop_attr, we[:], preferred_element_type=F32)
    xe = xlv + xr[:] + lee
    lr = jnp.maximum(xe, NEG * xe)
    a16 = jnp.exp(jnp.dot(lr, att16[:], preferred_element_type=F32))
    num = nm0[:] + nm1[:] + xlv * jnp.dot(a16, hexp[:],
                                          preferred_element_type=F32)
    den16 = dn0[:] + dn1[:] + a16
    den = jnp.dot(den16, hexp[:], preferred_element_type=F32)
    o = num / den + bias[:]
    out[:] = _ln(hv + o, g[:], b[:])


def _combine(h, xl, xr, numparts, denparts, dgparts, we, att16, hexp,
             bias, g, b):
    return pl.pallas_call(
        _combine_kernel,
        grid=(N // BN,),
        in_specs=[
            pl.BlockSpec((BN, HID), lambda i: (i, 0)),
            pl.BlockSpec((BN, HID), lambda i: (i, 0)),
            pl.BlockSpec((BN, HID), lambda i: (i, 0)),
            pl.BlockSpec((BN, HID), lambda i: (i, 0)),
            pl.BlockSpec((BN, HID), lambda i: (i, 0)),
            pl.BlockSpec((BN, 16), lambda i: (i, 0)),
            pl.BlockSpec((BN, 16), lambda i: (i, 0)),
            pl.BlockSpec((BN, 16), lambda i: (i, 0)),
            pl.BlockSpec((BN, 16), lambda i: (i, 0)),
            _full((N_EDGE, HID)), _full((HID, 16)), _full((16, HID)),
            _full((1, HID)), _full((1, HID)), _full((1, HID)),
        ],
        out_specs=pl.BlockSpec((BN, HID), lambda i: (i, 0)),
        out_shape=jax.ShapeDtypeStruct((N, HID), F32),
    )(h, xl, xr, numparts[0], numparts[1], denparts[0], denparts[1],
      dgparts[0], dgparts[1],
      we, att16, hexp, bias.reshape(1, -1), g.reshape(1, -1), b.reshape(1, -1))


def _heads_kernel(hs, hd, ea, m1a, m1b, m1c, m1bias, m2w, m2b,
                  f1a, f1b, f1c, f1bias, f2w, f2b, out):
    hsv = hs[:]
    hdv = hd[:]
    eav = ea[:]
    mh = _gelu(jnp.dot(hsv, m1a[:], preferred_element_type=F32)
               + jnp.dot(hdv, m1b[:], preferred_element_type=F32)
               + jnp.dot(eav, m1c[:], preferred_element_type=F32) + m1bias[:])
    ml = jnp.dot(mh, m2w[:], preferred_element_type=F32) + m2b[:]
    fh = _gelu(jnp.dot(hsv, f1a[:], preferred_element_type=F32)
               + jnp.dot(hdv, f1b[:], preferred_element_type=F32)
               + jnp.dot(eav, f1c[:], preferred_element_type=F32) + f1bias[:])
    fp = jnp.dot(fh, f2w[:], preferred_element_type=F32) + f2b[:]
    alpha = _softplus(fp[:, 0:1]) + 1e-4
    beta = _softplus(fp[:, 1:2]) + 1e-4
    out[:] = jnp.concatenate([ml, alpha, beta, jnp.zeros_like(ml)], axis=1)


def _heads(hs, hd, ea, p):
    return pl.pallas_call(
        _heads_kernel,
        grid=(E // BE,),
        in_specs=[
            pl.BlockSpec((BE, HID), lambda i: (i, 0)),
            pl.BlockSpec((BE, HID), lambda i: (i, 0)),
            pl.BlockSpec((BE, N_EDGE), lambda i: (i, 0)),
            _full((HID, HID)), _full((HID, HID)), _full((N_EDGE, HID)),
            _full((1, HID)), _full((HID, 1)), _full((1, 1)),
            _full((HID, HID)), _full((HID, HID)), _full((N_EDGE, HID)),
            _full((1, HID)), _full((HID, 2)), _full((1, 2)),
        ],
        out_specs=pl.BlockSpec((BE, 4), lambda i: (i, 0)),
        out_shape=jax.ShapeDtypeStruct((E, 4), F32),
    )(hs, hd, ea,
      p['m1_w'][:HID], p['m1_w'][HID:2 * HID], p['m1_w'][2 * HID:],
      p['m1_b'].reshape(1, -1), p['m2_w'], p['m2_b'].reshape(1, 1),
      p['f1_w'][:HID], p['f1_w'][HID:2 * HID], p['f1_w'][2 * HID:],
      p['f1_b'].reshape(1, -1), p['f2_w'], p['f2_b'].reshape(1, 2))


def _pool_kernel(h, mask, out):
    @pl.when(pl.program_id(0) == 0)
    def _init():
        out[:] = jnp.zeros_like(out)

    hv = h[:]
    mv = mask[:]
    s0 = jnp.sum(hv, axis=0, keepdims=True)
    s1 = jnp.sum(hv * mv, axis=0, keepdims=True)
    s2 = jnp.sum(mv, axis=0, keepdims=True) * jnp.ones((1, HID), F32)
    pad = jnp.zeros((5, HID), F32)
    out[:] += jnp.concatenate([s0, s1, s2, pad], axis=0)


def _pool(h, mask):
    return pl.pallas_call(
        _pool_kernel,
        grid=(N // BN,),
        in_specs=[
            pl.BlockSpec((BN, HID), lambda i: (i, 0)),
            pl.BlockSpec((BN, 1), lambda i: (i, 0)),
        ],
        out_specs=pl.BlockSpec((8, HID), lambda i: (0, 0)),
        out_shape=jax.ShapeDtypeStruct((8, HID), F32),
    )(h, mask)


def _value_kernel(sums, v1a, v1b, v1bias, v2w, v2b, out):
    inv_n = 1.0 / N
    gp = sums[0:1, :] * inv_n
    asum = sums[1:2, :] * inv_n
    acnt = jnp.maximum(sums[2:3, :] * inv_n, 1e-6)
    apool = asum / acnt
    vh = _gelu(jnp.dot(gp, v1a[:], preferred_element_type=F32)
               + jnp.dot(apool, v1b[:], preferred_element_type=F32) + v1bias[:])
    val = jnp.dot(vh, v2w[:], preferred_element_type=F32) + v2b[:]
    out[:] = val * jnp.ones((1, HID), F32)


def _value(sums, p):
    return pl.pallas_call(
        _value_kernel,
        grid=(1,),
        in_specs=[
            _full((8, HID)), _full((HID, HID)), _full((HID, HID)),
            _full((1, HID)), _full((HID, 1)), _full((1, 1)),
        ],
        out_specs=pl.BlockSpec((1, HID), lambda i: (0, 0)),
        out_shape=jax.ShapeDtypeStruct((1, HID), F32),
    )(sums, p['v1_w'][:HID], p['v1_w'][HID:], p['v1_b'].reshape(1, -1),
      p['v2_w'], p['v2_b'].reshape(1, 1))


def _att_mats(att):
    eye4 = jnp.eye(4, dtype=F32)
    att_d = (eye4[:, None, :] * att[:, :, None]).reshape(HID, 4)
    att16 = jnp.pad(att_d, ((0, 0), (0, 12)))
    return att16


_HEXP = None


def _hexp_mat():
    eye4 = jnp.eye(4, dtype=F32)
    return jnp.pad(jnp.repeat(eye4, C, axis=1), ((0, 12), (0, 0)))


def kernel(x, edge_index, edge_attr, u, acting_mask, params):
    p = params
    src2 = edge_index[0].reshape(NW, K, R)
    dst2 = edge_index[1].reshape(NW, K, R)
    mask_f = acting_mask.astype(F32).reshape(N, 1)
    uu = (u if u.ndim == 2 else u[None, :]).astype(F32)
    hexp = _hexp_mat()

    h = _encoder(x, uu, p)

    ea128 = jnp.concatenate(
        [edge_attr, jnp.ones((E, 1), F32), jnp.zeros((E, HID - 5), F32)],
        axis=1)
    dgparts = _scatter128(ea128, dst2)[:, :, :16]

    for i in range(4):
        cv = p['convs'][i]
        att16 = _att_mats(cv['att'])
        xl, xr = _lin2(h, cv['Wl'], cv['bl'], cv['Wr'], cv['br'])
        gl, gr = _gather2(xl, xr, src2, dst2)
        w_arr, a_arr = _edge_stage(gl, gr, edge_attr, cv['We'], att16, hexp)
        numparts = _scatter128(w_arr, dst2)
        denparts = _scatter128(a_arr, dst2)[:, :, :16]
        h = _combine(h, xl, xr, numparts, denparts, dgparts, cv['We'], att16,
                     hexp, cv['bias'], p['ln_g'][i], p['ln_b'][i])

    hs, hd = _gather2(h, h, src2, dst2)
    headout = _heads(hs, hd, edge_attr, p)
    sums = _pool(h, mask_f)
    valout = _value(sums, p)

    return headout[:, 0], headout[:, 1], headout[:, 2], valout[0, 0:1]


# speedup vs baseline: 40.5864x; 1.4000x over previous
import functools

import jax
import jax.numpy as jnp
from jax import lax
from jax.experimental import pallas as pl
from jax.experimental.pallas import tpu as pltpu
from jax.experimental.pallas import tpu_sc as plsc

F32 = jnp.float32

H = 4
C = 32
HID = 128
N_EDGE = 4
NEG = 0.2

N = 10000
E = 320000

NC = 2
NS = 16
NW = NC * NS
EW = E // NW
R = 80
K = EW // R

BN = 2000
BE = 2560


def _ln(x, g, b):
    m = jnp.mean(x, axis=-1, keepdims=True)
    v = jnp.mean((x - m) ** 2, axis=-1, keepdims=True)
    return (x - m) * jax.lax.rsqrt(v + 1e-5) * g + b


def _gelu(x):
    return 0.5 * x * (1.0 + lax.erf(x * 0.7071067811865476))


def _softplus(x):
    return jnp.maximum(x, 0.0) + jnp.log1p(jnp.exp(-jnp.abs(x)))


def _make_gather2():
    mesh = plsc.VectorSubcoreMesh(core_axis_name="c", subcore_axis_name="s")

    @functools.partial(
        pl.kernel,
        mesh=mesh,
        out_type=(
            jax.ShapeDtypeStruct((E, HID), F32),
            jax.ShapeDtypeStruct((E, HID), F32),
        ),
        scratch_types=[
            pltpu.VMEM((K, R), jnp.int32),
            pltpu.VMEM((K, R), jnp.int32),
            pltpu.VMEM((2, R, HID), F32),
            pltpu.VMEM((2, R, HID), F32),
        ] + [pltpu.SemaphoreType.DMA] * 8,
    )
    def gather2(tab_a, tab_b, idx_a, idx_b, out_a, out_b,
                idxa_v, idxb_v, buf_a, buf_b,
                ga0, ga1, gb0, gb1, sa0, sa1, sb0, sb1):
        cid = lax.axis_index("c")
        sid = lax.axis_index("s")
        wid = sid * NC + cid
        pltpu.sync_copy(idx_a.at[wid], idxa_v)
        pltpu.sync_copy(idx_b.at[wid], idxb_v)
        ga = (ga0, ga1)
        gb = (gb0, gb1)
        sa = (sa0, sa1)
        sb = (sb0, sb1)

        def issue_gathers(j, b):
            pltpu.async_copy(tab_a.at[idxa_v.at[j]], buf_a.at[b], ga[b])
            pltpu.async_copy(tab_b.at[idxb_v.at[j]], buf_b.at[b], gb[b])

        issue_gathers(0, 0)
        issue_gathers(1, 1)

        def step(j, b):
            e0 = wid * EW + j * R
            pltpu.make_async_copy(tab_a.at[idxa_v.at[j]], buf_a.at[b],
                                  ga[b]).wait()
            pltpu.async_copy(buf_a.at[b], out_a.at[pl.ds(e0, R), :], sa[b])
            pltpu.make_async_copy(tab_b.at[idxb_v.at[j]], buf_b.at[b],
                                  gb[b]).wait()
            pltpu.async_copy(buf_b.at[b], out_b.at[pl.ds(e0, R), :], sb[b])

            @pl.when(j + 2 < K)
            def _next():
                pltpu.make_async_copy(buf_a.at[b],
                                      out_a.at[pl.ds(e0, R), :], sa[b]).wait()
                pltpu.make_async_copy(buf_b.at[b],
                                      out_b.at[pl.ds(e0, R), :], sb[b]).wait()
                issue_gathers(j + 2, b)

        def pair(jj, carry):
            step(2 * jj, 0)
            step(2 * jj + 1, 1)
            return carry

        lax.fori_loop(0, K // 2, pair, 0)
        step(K - 1, (K - 1) % 2)

        for j in (K - 2, K - 1):
            b = j % 2
            e0 = wid * EW + j * R
            pltpu.make_async_copy(buf_a.at[b], out_a.at[pl.ds(e0, R), :],
                                  sa[b]).wait()
            pltpu.make_async_copy(buf_b.at[b], out_b.at[pl.ds(e0, R), :],
                                  sb[b]).wait()

    return gather2


def _make_scatter(width):
    mesh = plsc.VectorSubcoreMesh(core_axis_name="c", subcore_axis_name="s")
    rps = 624
    tail0 = rps * NS

    @functools.partial(
        pl.kernel,
        mesh=mesh,
        out_type=jax.ShapeDtypeStruct((NC, N, width), F32),
        scratch_types=[
            pltpu.VMEM((K, R), jnp.int32),
            pltpu.VMEM((2, R, width), F32),
            pltpu.VMEM_SHARED((N, width), F32),
        ] + [pltpu.SemaphoreType.DMA] * 4,
    )
    def scatter(vals, idx, zeros, out, idx_v, buf, acc, ld0, ld1, sc0, sc1):
        cid = lax.axis_index("c")
        sid = lax.axis_index("s")
        wid = sid * NC + cid
        r0 = sid * rps
        pltpu.sync_copy(zeros.at[pl.ds(r0, rps), :],
                        acc.at[pl.ds(r0, rps), :])

        @pl.when(sid == NS - 1)
        def _init_tail():
            pltpu.sync_copy(zeros.at[pl.ds(tail0, N - tail0), :],
                            acc.at[pl.ds(tail0, N - tail0), :])

        plsc.subcore_barrier()
        pltpu.sync_copy(idx.at[wid], idx_v)
        ld = (ld0, ld1)
        sc = (sc0, sc1)

        def issue_load(j, b):
            e0 = wid * EW + j * R
            pltpu.async_copy(vals.at[pl.ds(e0, R), :], buf.at[b], ld[b])

        issue_load(0, 0)
        issue_load(1, 1)

        def chunk(j, b):
            e0 = wid * EW + j * R
            pltpu.make_async_copy(vals.at[pl.ds(e0, R), :], buf.at[b],
                                  ld[b]).wait()
            pltpu.async_copy(buf.at[b], acc.at[idx_v.at[j]], sc[b], add=True)

            @pl.when(j + 2 < K)
            def _next():
                pltpu.make_async_copy(buf.at[b], acc.at[idx_v.at[j]],
                                      sc[b]).wait()
                issue_load(j + 2, b)

        def pair(jj, carry):
            chunk(2 * jj, 0)
            chunk(2 * jj + 1, 1)
            return carry

        lax.fori_loop(0, K // 2, pair, 0)
        chunk(K - 1, (K - 1) % 2)
        for j in (K - 2, K - 1):
            b = j % 2
            pltpu.make_async_copy(buf.at[b], acc.at[idx_v.at[j]],
                                  sc[b]).wait()
        plsc.subcore_barrier()
        pltpu.sync_copy(acc.at[pl.ds(r0, rps), :],
                        out.at[cid, pl.ds(r0, rps), :])

        @pl.when(sid == NS - 1)
        def _drain_tail():
            pltpu.sync_copy(acc.at[pl.ds(tail0, N - tail0), :],
                            out.at[cid, pl.ds(tail0, N - tail0), :])

    return scatter


_SC_CACHE = {}


def _gather2(tab_a, tab_b, idx_a, idx_b):
    if 'g2' not in _SC_CACHE:
        _SC_CACHE['g2'] = _make_gather2()
    return _SC_CACHE['g2'](tab_a, tab_b, idx_a, idx_b)


def _scatter128(vals, idx):
    if 's128' not in _SC_CACHE:
        _SC_CACHE['s128'] = _make_scatter(HID)
    return _SC_CACHE['s128'](vals, idx, jnp.zeros((N, HID), F32))


def _full(shape):
    nd = len(shape)
    return pl.BlockSpec(shape, lambda i: (0,) * nd)


def _enc_kernel(x, uu, ne_w, ne_b, ne_g, ne_bt, ge_w, ge_b, ge_g, ge_bt, out):
    t = jnp.dot(x[:], ne_w[:], preferred_element_type=F32) + ne_b[:]
    t = _gelu(_ln(t, ne_g[:], ne_bt[:]))
    tu = jnp.dot(uu[:], ge_w[:], preferred_element_type=F32) + ge_b[:]
    tu = _gelu(_ln(tu, ge_g[:], ge_bt[:]))
    out[:] = t + tu


def _encoder(x, uu, p):
    return pl.pallas_call(
        _enc_kernel,
        grid=(N // BN,),
        in_specs=[
            pl.BlockSpec((BN, 18), lambda i: (i, 0)),
            _full((1, 8)), _full((18, HID)), _full((1, HID)), _full((1, HID)),
            _full((1, HID)), _full((8, HID)), _full((1, HID)), _full((1, HID)),
            _full((1, HID)),
        ],
        out_specs=pl.BlockSpec((BN, HID), lambda i: (i, 0)),
        out_shape=jax.ShapeDtypeStruct((N, HID), F32),
    )(x, uu, p['ne_w'], p['ne_b'].reshape(1, -1), p['ne_g'].reshape(1, -1),
      p['ne_bt'].reshape(1, -1), p['ge_w'], p['ge_b'].reshape(1, -1),
      p['ge_g'].reshape(1, -1), p['ge_bt'].reshape(1, -1))


def _lin2_kernel(h, wl, bl, wr, br, xl, xr):
    hh = h[:]
    xl[:] = jnp.dot(hh, wl[:], preferred_element_type=F32) + bl[:]
    xr[:] = jnp.dot(hh, wr[:], preferred_element_type=F32) + br[:]


def _lin2(h, wl, bl, wr, br):
    return pl.pallas_call(
        _lin2_kernel,
        grid=(N // BN,),
        in_specs=[
            pl.BlockSpec((BN, HID), lambda i: (i, 0)),
            _full((HID, HID)), _full((1, HID)),
            _full((HID, HID)), _full((1, HID)),
        ],
        out_specs=[
            pl.BlockSpec((BN, HID), lambda i: (i, 0)),
            pl.BlockSpec((BN, HID), lambda i: (i, 0)),
        ],
        out_shape=[
            jax.ShapeDtypeStruct((N, HID), F32),
            jax.ShapeDtypeStruct((N, HID), F32),
        ],
    )(h, wl, bl.reshape(1, -1), wr, br.reshape(1, -1))


def _edge_kernel(gl, gr, ea, we, att16, hexp, w_out, a_out):
    glv = gl[:]
    ee = jnp.dot(ea[:], we[:], preferred_element_type=F32)
    xe = glv + gr[:] + ee
    lr = jnp.maximum(xe, NEG * xe)
    a16 = jnp.exp(jnp.dot(lr, att16[:], preferred_element_type=F32))
    w_out[:] = glv * jnp.dot(a16, hexp[:], preferred_element_type=F32)
    a_out[:] = jnp.concatenate([a16, jnp.zeros((BE, HID - 16), F32)], axis=1)


def _edge_stage(gl, gr, ea, we, att16, hexp):
    return pl.pallas_call(
        _edge_kernel,
        grid=(E // BE,),
        in_specs=[
            pl.BlockSpec((BE, HID), lambda i: (i, 0)),
            pl.BlockSpec((BE, HID), lambda i: (i, 0)),
            pl.BlockSpec((BE, N_EDGE), lambda i: (i, 0)),
            _full((N_EDGE, HID)), _full((HID, 16)), _full((16, HID)),
        ],
        out_specs=[
            pl.BlockSpec((BE, HID), lambda i: (i, 0)),
            pl.BlockSpec((BE, HID), lambda i: (i, 0)),
        ],
        out_shape=[
            jax.ShapeDtypeStruct((E, HID), F32),
            jax.ShapeDtypeStruct((E, HID), F32),
        ],
    )(gl, gr, ea, we, att16, hexp)


def _combine_kernel(h, xl, xr, nm0, nm1, dn0, dn1, dg0, dg1, we, att16, hexp,
                    bias, g, b, out):
    hv = h[:]
    xlv = xl[:]
    dl = dg0[:] + dg1[:]
    deg = jnp.maximum(dl[:, 4:5], 1.0)
    loop_attr = dl[:, 0:4] / deg
    lee = jnp.dot(loop_attr, we[:], preferred_element_type=F32)
    xe = xlv + xr[:] + lee
    lr = jnp.maximum(xe, NEG * xe)
    a16 = jnp.exp(jnp.dot(lr, att16[:], preferred_element_type=F32))
    num = nm0[:] + nm1[:] + xlv * jnp.dot(a16, hexp[:],
                                          preferred_element_type=F32)
    den16 = dn0[:] + dn1[:] + a16
    den = jnp.dot(den16, hexp[:], preferred_element_type=F32)
    o = num / den + bias[:]
    out[:] = _ln(hv + o, g[:], b[:])


def _combine(h, xl, xr, numparts, denparts, dgparts, we, att16, hexp,
             bias, g, b):
    return pl.pallas_call(
        _combine_kernel,
        grid=(N // BN,),
        in_specs=[
            pl.BlockSpec((BN, HID), lambda i: (i, 0)),
            pl.BlockSpec((BN, HID), lambda i: (i, 0)),
            pl.BlockSpec((BN, HID), lambda i: (i, 0)),
            pl.BlockSpec((BN, HID), lambda i: (i, 0)),
            pl.BlockSpec((BN, HID), lambda i: (i, 0)),
            pl.BlockSpec((BN, 16), lambda i: (i, 0)),
            pl.BlockSpec((BN, 16), lambda i: (i, 0)),
            pl.BlockSpec((BN, 16), lambda i: (i, 0)),
            pl.BlockSpec((BN, 16), lambda i: (i, 0)),
            _full((N_EDGE, HID)), _full((HID, 16)), _full((16, HID)),
            _full((1, HID)), _full((1, HID)), _full((1, HID)),
        ],
        out_specs=pl.BlockSpec((BN, HID), lambda i: (i, 0)),
        out_shape=jax.ShapeDtypeStruct((N, HID), F32),
    )(h, xl, xr, numparts[0], numparts[1], denparts[0], denparts[1],
      dgparts[0], dgparts[1],
      we, att16, hexp, bias.reshape(1, -1), g.reshape(1, -1), b.reshape(1, -1))


def _heads_kernel(hs, hd, ea, m1a, m1b, m1c, m1bias, m2w, m2b,
                  f1a, f1b, f1c, f1bias, f2w, f2b, out):
    hsv = hs[:]
    hdv = hd[:]
    eav = ea[:]
    mh = _gelu(jnp.dot(hsv, m1a[:], preferred_element_type=F32)
               + jnp.dot(hdv, m1b[:], preferred_element_type=F32)
               + jnp.dot(eav, m1c[:], preferred_element_type=F32) + m1bias[:])
    ml = jnp.dot(mh, m2w[:], preferred_element_type=F32) + m2b[:]
    fh = _gelu(jnp.dot(hsv, f1a[:], preferred_element_type=F32)
               + jnp.dot(hdv, f1b[:], preferred_element_type=F32)
               + jnp.dot(eav, f1c[:], preferred_element_type=F32) + f1bias[:])
    fp = jnp.dot(fh, f2w[:], preferred_element_type=F32) + f2b[:]
    alpha = _softplus(fp[:, 0:1]) + 1e-4
    beta = _softplus(fp[:, 1:2]) + 1e-4
    out[:] = jnp.concatenate([ml, alpha, beta, jnp.zeros_like(ml)], axis=1)


def _heads(hs, hd, ea, p):
    return pl.pallas_call(
        _heads_kernel,
        grid=(E // BE,),
        in_specs=[
            pl.BlockSpec((BE, HID), lambda i: (i, 0)),
            pl.BlockSpec((BE, HID), lambda i: (i, 0)),
            pl.BlockSpec((BE, N_EDGE), lambda i: (i, 0)),
            _full((HID, HID)), _full((HID, HID)), _full((N_EDGE, HID)),
            _full((1, HID)), _full((HID, 1)), _full((1, 1)),
            _full((HID, HID)), _full((HID, HID)), _full((N_EDGE, HID)),
            _full((1, HID)), _full((HID, 2)), _full((1, 2)),
        ],
        out_specs=pl.BlockSpec((BE, 4), lambda i: (i, 0)),
        out_shape=jax.ShapeDtypeStruct((E, 4), F32),
    )(hs, hd, ea,
      p['m1_w'][:HID], p['m1_w'][HID:2 * HID], p['m1_w'][2 * HID:],
      p['m1_b'].reshape(1, -1), p['m2_w'], p['m2_b'].reshape(1, 1),
      p['f1_w'][:HID], p['f1_w'][HID:2 * HID], p['f1_w'][2 * HID:],
      p['f1_b'].reshape(1, -1), p['f2_w'], p['f2_b'].reshape(1, 2))


def _pool_kernel(h, mask, out):
    @pl.when(pl.program_id(0) == 0)
    def _init():
        out[:] = jnp.zeros_like(out)

    hv = h[:]
    mv = mask[:]
    s0 = jnp.sum(hv, axis=0, keepdims=True)
    s1 = jnp.sum(hv * mv, axis=0, keepdims=True)
    s2 = jnp.sum(mv, axis=0, keepdims=True) * jnp.ones((1, HID), F32)
    pad = jnp.zeros((5, HID), F32)
    out[:] += jnp.concatenate([s0, s1, s2, pad], axis=0)


def _pool(h, mask):
    return pl.pallas_call(
        _pool_kernel,
        grid=(N // BN,),
        in_specs=[
            pl.BlockSpec((BN, HID), lambda i: (i, 0)),
            pl.BlockSpec((BN, 1), lambda i: (i, 0)),
        ],
        out_specs=pl.BlockSpec((8, HID), lambda i: (0, 0)),
        out_shape=jax.ShapeDtypeStruct((8, HID), F32),
    )(h, mask)


def _value_kernel(sums, v1a, v1b, v1bias, v2w, v2b, out):
    inv_n = 1.0 / N
    gp = sums[0:1, :] * inv_n
    asum = sums[1:2, :] * inv_n
    acnt = jnp.maximum(sums[2:3, :] * inv_n, 1e-6)
    apool = asum / acnt
    vh = _gelu(jnp.dot(gp, v1a[:], preferred_element_type=F32)
               + jnp.dot(apool, v1b[:], preferred_element_type=F32) + v1bias[:])
    val = jnp.dot(vh, v2w[:], preferred_element_type=F32) + v2b[:]
    out[:] = val * jnp.ones((1, HID), F32)


def _value(sums, p):
    return pl.pallas_call(
        _value_kernel,
        grid=(1,),
        in_specs=[
            _full((8, HID)), _full((HID, HID)), _full((HID, HID)),
            _full((1, HID)), _full((HID, 1)), _full((1, 1)),
        ],
        out_specs=pl.BlockSpec((1, HID), lambda i: (0, 0)),
        out_shape=jax.ShapeDtypeStruct((1, HID), F32),
    )(sums, p['v1_w'][:HID], p['v1_w'][HID:], p['v1_b'].reshape(1, -1),
      p['v2_w'], p['v2_b'].reshape(1, 1))


def _att_mats(att):
    eye4 = jnp.eye(4, dtype=F32)
    att_d = (eye4[:, None, :] * att[:, :, None]).reshape(HID, 4)
    att16 = jnp.pad(att_d, ((0, 0), (0, 12)))
    return att16


_HEXP = None


def _hexp_mat():
    eye4 = jnp.eye(4, dtype=F32)
    return jnp.pad(jnp.repeat(eye4, C, axis=1), ((0, 12), (0, 0)))


def kernel(x, edge_index, edge_attr, u, acting_mask, params):
    p = params
    src2 = edge_index[0].reshape(NW, K, R)
    dst2 = edge_index[1].reshape(NW, K, R)
    mask_f = acting_mask.astype(F32).reshape(N, 1)
    uu = (u if u.ndim == 2 else u[None, :]).astype(F32)
    hexp = _hexp_mat()

    h = _encoder(x, uu, p)

    ea128 = jnp.concatenate(
        [edge_attr, jnp.ones((E, 1), F32), jnp.zeros((E, HID - 5), F32)],
        axis=1)
    dgparts = _scatter128(ea128, dst2)[:, :, :16]

    for i in range(4):
        cv = p['convs'][i]
        att16 = _att_mats(cv['att'])
        xl, xr = _lin2(h, cv['Wl'], cv['bl'], cv['Wr'], cv['br'])
        gl, gr = _gather2(xl, xr, src2, dst2)
        w_arr, a_arr = _edge_stage(gl, gr, edge_attr, cv['We'], att16, hexp)
        numparts = _scatter128(w_arr, dst2)
        denparts = _scatter128(a_arr, dst2)[:, :, :16]
        h = _combine(h, xl, xr, numparts, denparts, dgparts, cv['We'], att16,
                     hexp, cv['bias'], p['ln_g'][i], p['ln_b'][i])

    hs, hd = _gather2(h, h, src2, dst2)
    headout = _heads(hs, hd, edge_attr, p)
    sums = _pool(h, mask_f)
    valout = _value(sums, p)

    return headout[:, 0], headout[:, 1], headout[:, 2], valout[0, 0:1]
